# baseline (speedup 1.0000x reference)
import functools

import jax
import jax.numpy as jnp
from jax import lax
from jax.experimental import pallas as pl
from jax.experimental.pallas import tpu as pltpu
from jax.experimental.pallas import tpu_sc as plsc

DEPTH = 1000
BATCH = 16384

NC = 2
NS = 16
L = 16
NW = NC * NS
SPW = BATCH // NW
CR = 40
NCH = DEPTH // CR

_mesh = plsc.VectorSubcoreMesh(core_axis_name="c", subcore_axis_name="s")


@functools.partial(
    pl.kernel,
    out_type=jax.ShapeDtypeStruct((DEPTH, BATCH), jnp.float32),
    mesh=_mesh,
    scratch_types=[
        pltpu.VMEM((SPW,), jnp.int32),
        pltpu.VMEM((CR, SPW), jnp.float32),
        pltpu.VMEM((CR, SPW), jnp.float32),
        pltpu.SemaphoreType.DMA,
        pltpu.SemaphoreType.DMA,
    ],
    compiler_params=pltpu.CompilerParams(
        needs_layout_passes=False, skip_device_barrier=True
    ),
)
def _onehot_t_sc(x_hbm, out_hbm, idx_v, buf_a, buf_b, sem_a, sem_b):
    wid = lax.axis_index("s") * NC + lax.axis_index("c")
    base = wid * SPW

    pltpu.sync_copy(x_hbm.at[pl.ds(base, SPW)], idx_v)

    zeros16 = jnp.zeros((L,), jnp.float32)
    ones16 = jnp.ones((L,), jnp.float32)
    lane = lax.iota(jnp.int32, L)

    bufs = (buf_a, buf_b)
    sems = (sem_a, sem_b)

    def zero_buf(buf):
        def zero_rows(i, _):
            for o in range(0, SPW, L):
                buf[i, pl.ds(o, L)] = zeros16
            return 0

        lax.fori_loop(0, CR, zero_rows, 0)

    def put(buf, row0, vals):
        def group(g, _):
            idxs = idx_v[pl.ds(g * L, L)]
            rows = (idxs - row0).astype(jnp.uint32)
            mask = rows < CR
            rows = jnp.minimum(rows, CR - 1).astype(jnp.int32)
            plsc.store_scatter(buf, [rows, g * L + lane], vals, mask=mask)
            return 0

        lax.fori_loop(0, SPW // L, group, 0)

    def fire(buf, sem, c):
        put(buf, c * CR, ones16)
        pltpu.async_copy(
            buf, out_hbm.at[pl.ds(c * CR, CR), pl.ds(base, SPW)], sem
        )

    def drain_one(buf, sem):
        pltpu.make_async_copy(
            buf, out_hbm.at[pl.ds(0, CR), pl.ds(base, SPW)], sem
        ).wait()

    zero_buf(buf_a)
    fire(buf_a, sem_a, 0)
    zero_buf(buf_b)
    fire(buf_b, sem_b, 1)

    def step(c, _):
        for b in range(2):
            @pl.when(c % 2 == b)
            def _():
                buf, sem = bufs[b], sems[b]
                drain_one(buf, sem)
                put(buf, (c - 2) * CR, zeros16)
                fire(buf, sem, c)
        return 0

    lax.fori_loop(2, NCH, step, 0)
    drain_one(buf_a, sem_a)
    drain_one(buf_b, sem_b)


def kernel(X_in, ones):
    del ones
    return _onehot_t_sc(X_in.astype(jnp.int32)).T


# speedup vs baseline: 1.0141x; 1.0141x over previous
import functools

import jax
import jax.numpy as jnp
from jax import lax
from jax.experimental import pallas as pl
from jax.experimental.pallas import tpu as pltpu
from jax.experimental.pallas import tpu_sc as plsc

DEPTH = 1000
BATCH = 16384

NC = 2
NS = 16
L = 16
NW = NC * NS
SPW = BATCH // NW
CR = 40
NCH = DEPTH // CR

_mesh = plsc.VectorSubcoreMesh(core_axis_name="c", subcore_axis_name="s")


@functools.partial(
    pl.kernel,
    out_type=jax.ShapeDtypeStruct((DEPTH, BATCH), jnp.float32),
    mesh=_mesh,
    scratch_types=[
        pltpu.VMEM((SPW,), jnp.int32),
        pltpu.VMEM((CR, SPW), jnp.float32),
        pltpu.VMEM((CR, SPW), jnp.float32),
        pltpu.SemaphoreType.DMA,
        pltpu.SemaphoreType.DMA,
    ],
    compiler_params=pltpu.CompilerParams(needs_layout_passes=False),
)
def _onehot_t_sc(x_hbm, out_hbm, idx_v, buf_a, buf_b, sem_a, sem_b):
    wid = lax.axis_index("s") * NC + lax.axis_index("c")
    base = wid * SPW

    idx_cp = pltpu.async_copy(x_hbm.at[pl.ds(base, SPW)], idx_v, sem_a)

    zeros16 = jnp.zeros((L,), jnp.float32)
    ones16 = jnp.ones((L,), jnp.float32)
    lane = lax.iota(jnp.int32, L)

    bufs = (buf_a, buf_b)
    sems = (sem_a, sem_b)

    def zero_buf(buf):
        def zero_rows(i, _):
            for o in range(0, SPW, L):
                buf[i, pl.ds(o, L)] = zeros16
            return 0

        lax.fori_loop(0, CR, zero_rows, 0)

    def put(buf, row0, vals):
        def group(g, _):
            idxs = idx_v[pl.ds(g * L, L)]
            rows = (idxs - row0).astype(jnp.uint32)
            mask = rows < CR
            rows = jnp.minimum(rows, CR - 1).astype(jnp.int32)
            plsc.store_scatter(buf, [rows, g * L + lane], vals, mask=mask)
            return 0

        lax.fori_loop(0, SPW // L, group, 0)

    def fire(buf, sem, c):
        put(buf, c * CR, ones16)
        pltpu.async_copy(
            buf, out_hbm.at[pl.ds(c * CR, CR), pl.ds(base, SPW)], sem
        )

    def drain_one(buf, sem):
        pltpu.make_async_copy(
            buf, out_hbm.at[pl.ds(0, CR), pl.ds(base, SPW)], sem
        ).wait()

    zero_buf(buf_a)
    idx_cp.wait()
    fire(buf_a, sem_a, 0)
    zero_buf(buf_b)
    fire(buf_b, sem_b, 1)

    def step(c, _):
        for b in range(2):
            @pl.when(c % 2 == b)
            def _():
                buf, sem = bufs[b], sems[b]
                drain_one(buf, sem)
                put(buf, (c - 2) * CR, zeros16)
                fire(buf, sem, c)
        return 0

    lax.fori_loop(2, NCH, step, 0)
    drain_one(buf_a, sem_a)
    drain_one(buf_b, sem_b)


def kernel(X_in, ones):
    del ones
    return _onehot_t_sc(X_in.astype(jnp.int32)).T
